# Initial kernel scaffold; baseline (speedup 1.0000x reference)
import functools

import jax
import jax.numpy as jnp
from jax import lax
from jax.experimental import pallas as pl
from jax.experimental.pallas import tpu as pltpu
from jax.experimental.pallas import tpu_sc as plsc

N_PAD = 10240
NC, NS, NW = 2, 16, 32
CH = 80
NCH = 125
RPS = N_PAD // NS

_MESH = plsc.VectorSubcoreMesh(
    core_axis_name="c", subcore_axis_name="s", num_cores=NC, num_subcores=NS)


def _make_deg_kernel():
  @functools.partial(
      pl.kernel, mesh=_MESH,
      out_type=jax.ShapeDtypeStruct((NC, N_PAD, 16), jnp.float32),
      scratch_types=[
          pltpu.VMEM((NCH, CH), jnp.int32),
          pltpu.VMEM((CH, 16), jnp.float32),
          pltpu.VMEM_SHARED((N_PAD, 16), jnp.float32),
      ],
  )
  def deg_kernel(dst_hbm, zeros_hbm, out_hbm, idx_v, ones_v, deg_sh):
    cid = lax.axis_index("c")
    sid = lax.axis_index("s")
    wid = cid * NS + sid
    pltpu.sync_copy(zeros_hbm.at[:, 0:16], deg_sh.at[pl.ds(sid * RPS, RPS)])
    pltpu.sync_copy(dst_hbm.at[wid], idx_v)
    for r in range(CH):
      ones_v[r, :] = jnp.full((16,), 1.0, jnp.float32)
    plsc.subcore_barrier()

    def body(c, carry):
      pltpu.sync_copy(ones_v, deg_sh.at[idx_v.at[c]], add=True)
      return carry

    lax.fori_loop(0, NCH, body, 0)
    plsc.subcore_barrier()
    pltpu.sync_copy(deg_sh.at[pl.ds(sid * RPS, RPS)],
                    out_hbm.at[cid, pl.ds(sid * RPS, RPS)])

  return deg_kernel


def _make_edge_kernel(d):
  @functools.partial(
      pl.kernel, mesh=_MESH,
      out_type=jax.ShapeDtypeStruct((NC, N_PAD, d), jnp.float32),
      scratch_types=[
          pltpu.VMEM((NCH, CH), jnp.int32),
          pltpu.VMEM((NCH, CH), jnp.int32),
          pltpu.VMEM((CH, d), jnp.float32),
          pltpu.VMEM_SHARED((N_PAD, d), jnp.float32),
      ],
  )
  def edge_kernel(y_hbm, src_hbm, dst_hbm, zeros_hbm, out_hbm,
                  src_v, dst_v, rows_v, acc_sh):
    cid = lax.axis_index("c")
    sid = lax.axis_index("s")
    wid = cid * NS + sid
    pltpu.sync_copy(zeros_hbm.at[:, 0:d], acc_sh.at[pl.ds(sid * RPS, RPS)])
    pltpu.sync_copy(src_hbm.at[wid], src_v)
    pltpu.sync_copy(dst_hbm.at[wid], dst_v)
    plsc.subcore_barrier()

    def body(c, carry):
      pltpu.sync_copy(y_hbm.at[src_v.at[c]], rows_v)
      pltpu.sync_copy(rows_v, acc_sh.at[dst_v.at[c]], add=True)
      return carry

    lax.fori_loop(0, NCH, body, 0)
    plsc.subcore_barrier()
    pltpu.sync_copy(acc_sh.at[pl.ds(sid * RPS, RPS)],
                    out_hbm.at[cid, pl.ds(sid * RPS, RPS)])

  return edge_kernel


def _make_decode_kernel(d):
  @functools.partial(
      pl.kernel, mesh=_MESH,
      out_type=jax.ShapeDtypeStruct((NW, NCH, CH), jnp.float32),
      scratch_types=[
          pltpu.VMEM((NCH, CH), jnp.int32),
          pltpu.VMEM((NCH, CH), jnp.int32),
          pltpu.VMEM((CH, d), jnp.float32),
          pltpu.VMEM((CH, d), jnp.float32),
          pltpu.VMEM((NCH, CH), jnp.float32),
      ],
  )
  def decode_kernel(z_hbm, src_hbm, dst_hbm, out_hbm,
                    src_v, dst_v, zr_v, zc_v, out_v):
    cid = lax.axis_index("c")
    sid = lax.axis_index("s")
    wid = cid * NS + sid
    pltpu.sync_copy(src_hbm.at[wid], src_v)
    pltpu.sync_copy(dst_hbm.at[wid], dst_v)

    def body(c, carry):
      pltpu.sync_copy(z_hbm.at[src_v.at[c]], zr_v)
      pltpu.sync_copy(z_hbm.at[dst_v.at[c]], zc_v)
      for g in range(CH // 16):
        rows = lax.iota(jnp.int32, 16) + (g * 16)
        acc = jnp.zeros((16,), jnp.float32)
        for j in range(d):
          col = jnp.full((16,), j, jnp.int32)
          acc = acc + (plsc.load_gather(zr_v, [rows, col]) *
                       plsc.load_gather(zc_v, [rows, col]))
        out_v[c, pl.ds(g * 16, 16)] = acc
      return carry

    lax.fori_loop(0, NCH, body, 0)
    pltpu.sync_copy(out_v, out_hbm.at[wid])

  return decode_kernel


def _tc_y1(x_pad, W1, degp):
  blk = 1024
  grid = N_PAD // blk

  def body(xb, wb, degb, yb):
    deg = degb[0] + degb[1] + 1.0
    dinv = lax.rsqrt(deg).reshape(blk, 1)
    yb[...] = dinv * jnp.dot(xb[...], wb[...],
                             preferred_element_type=jnp.float32)

  return pl.pallas_call(
      body,
      grid=(grid,),
      in_specs=[
          pl.BlockSpec((blk, 128), lambda i: (i, 0)),
          pl.BlockSpec((128, 128), lambda i: (0, 0)),
          pl.BlockSpec((2, blk // 128, 128), lambda i: (0, i, 0)),
      ],
      out_specs=pl.BlockSpec((blk, 128), lambda i: (i, 0)),
      out_shape=jax.ShapeDtypeStruct((N_PAD, 128), jnp.float32),
  )(x_pad, W1, degp)


def _tc_layer2_in(S1p, y1, degp, b1, W2):
  blk = 1024
  grid = N_PAD // blk

  def body(sb, yb, degb, bb, wb, ob):
    deg = degb[0] + degb[1] + 1.0
    dinv = lax.rsqrt(deg).reshape(blk, 1)
    h = jax.nn.relu(dinv * (sb[0] + sb[1] + yb[...]) + bb[...])
    ob[...] = dinv * jnp.dot(h, wb[...], preferred_element_type=jnp.float32)

  return pl.pallas_call(
      body,
      grid=(grid,),
      in_specs=[
          pl.BlockSpec((2, blk, 128), lambda i: (0, i, 0)),
          pl.BlockSpec((blk, 128), lambda i: (i, 0)),
          pl.BlockSpec((2, blk // 128, 128), lambda i: (0, i, 0)),
          pl.BlockSpec((1, 128), lambda i: (0, 0)),
          pl.BlockSpec((128, 64), lambda i: (0, 0)),
      ],
      out_specs=pl.BlockSpec((blk, 64), lambda i: (i, 0)),
      out_shape=jax.ShapeDtypeStruct((N_PAD, 64), jnp.float32),
  )(S1p, y1, degp, b1, W2)


def _tc_z(S2p, y2, degp, b2):
  blk = 1024
  grid = N_PAD // blk

  def body(sb, yb, degb, bb, zb):
    deg = degb[0] + degb[1] + 1.0
    dinv = lax.rsqrt(deg).reshape(blk, 1)
    zb[...] = dinv * (sb[0] + sb[1] + yb[...]) + bb[...]

  return pl.pallas_call(
      body,
      grid=(grid,),
      in_specs=[
          pl.BlockSpec((2, blk, 64), lambda i: (0, i, 0)),
          pl.BlockSpec((blk, 64), lambda i: (i, 0)),
          pl.BlockSpec((2, blk // 128, 128), lambda i: (0, i, 0)),
          pl.BlockSpec((1, 64), lambda i: (0, 0)),
      ],
      out_specs=pl.BlockSpec((blk, 64), lambda i: (i, 0)),
      out_shape=jax.ShapeDtypeStruct((N_PAD, 64), jnp.float32),
  )(S2p, y2, degp, b2)


_deg_kernel = _make_deg_kernel()
_edge_kernel_128 = _make_edge_kernel(128)
_edge_kernel_64 = _make_edge_kernel(64)
_decode_kernel = _make_decode_kernel(64)


def kernel(x, edge_index, W1, b1, W2, b2):
  n, d_in = x.shape
  e = edge_index.shape[1]
  x_pad = jnp.pad(x, ((0, N_PAD - n), (0, 0)))
  src_r = edge_index[0].reshape(NW, NCH, CH)
  dst_r = edge_index[1].reshape(NW, NCH, CH)
  zeros_init = jnp.zeros((RPS, 128), jnp.float32)

  deg_part = _deg_kernel(dst_r, zeros_init)
  degp = deg_part[:, :, 0].reshape(NC, N_PAD // 128, 128)

  y1 = _tc_y1(x_pad, W1, degp)
  S1p = _edge_kernel_128(y1, src_r, dst_r, zeros_init)
  y2 = _tc_layer2_in(S1p, y1, degp, b1.reshape(1, 128), W2)
  S2p = _edge_kernel_64(y2, src_r, dst_r, zeros_init)
  z = _tc_z(S2p, y2, degp, b2.reshape(1, 64))

  scores = _decode_kernel(z, src_r, dst_r)
  return scores.reshape(e)


# speedup vs baseline: 8.6915x; 8.6915x over previous
import functools

import jax
import jax.numpy as jnp
from jax import lax
from jax.experimental import pallas as pl
from jax.experimental.pallas import tpu as pltpu
from jax.experimental.pallas import tpu_sc as plsc

N_PAD = 10240
NC, NS, NW = 2, 16, 32
CH = 80
NCH = 125
RPS = N_PAD // NS

_MESH = plsc.VectorSubcoreMesh(
    core_axis_name="c", subcore_axis_name="s", num_cores=NC, num_subcores=NS)


def _make_deg_kernel():
  @functools.partial(
      pl.kernel, mesh=_MESH,
      compiler_params=pltpu.CompilerParams(use_tc_tiling_on_sc=False, needs_layout_passes=False),
      out_type=jax.ShapeDtypeStruct((NC, N_PAD, 16), jnp.float32),
      scratch_types=[
          pltpu.VMEM((NCH, CH), jnp.int32),
          pltpu.VMEM((CH, 16), jnp.float32),
          pltpu.VMEM_SHARED((N_PAD, 16), jnp.float32),
      ],
  )
  def deg_kernel(dst_hbm, zeros_hbm, out_hbm, idx_v, ones_v, deg_sh):
    cid = lax.axis_index("c")
    sid = lax.axis_index("s")
    wid = cid * NS + sid
    pltpu.sync_copy(zeros_hbm, deg_sh.at[pl.ds(sid * RPS, RPS)])
    pltpu.sync_copy(dst_hbm.at[wid], idx_v)
    for r in range(CH):
      ones_v[r, :] = jnp.full((16,), 1.0, jnp.float32)
    plsc.subcore_barrier()

    def body(c, carry):
      pltpu.sync_copy(ones_v, deg_sh.at[idx_v.at[c]], add=True)
      return carry

    lax.fori_loop(0, NCH, body, 0)
    plsc.subcore_barrier()
    pltpu.sync_copy(deg_sh.at[pl.ds(sid * RPS, RPS)],
                    out_hbm.at[cid, pl.ds(sid * RPS, RPS)])

  return deg_kernel


def _make_edge_kernel(d):
  @functools.partial(
      pl.kernel, mesh=_MESH,
      compiler_params=pltpu.CompilerParams(use_tc_tiling_on_sc=False, needs_layout_passes=False),
      out_type=jax.ShapeDtypeStruct((NC, N_PAD, d), jnp.float32),
      scratch_types=[
          pltpu.VMEM((NCH, CH), jnp.int32),
          pltpu.VMEM((NCH, CH), jnp.int32),
          pltpu.VMEM((CH, d), jnp.float32),
          pltpu.VMEM_SHARED((N_PAD, d), jnp.float32),
      ],
  )
  def edge_kernel(y_hbm, src_hbm, dst_hbm, zeros_hbm, out_hbm,
                  src_v, dst_v, rows_v, acc_sh):
    cid = lax.axis_index("c")
    sid = lax.axis_index("s")
    wid = cid * NS + sid
    pltpu.sync_copy(zeros_hbm, acc_sh.at[pl.ds(sid * RPS, RPS)])
    pltpu.sync_copy(src_hbm.at[wid], src_v)
    pltpu.sync_copy(dst_hbm.at[wid], dst_v)
    plsc.subcore_barrier()

    def body(c, carry):
      pltpu.sync_copy(y_hbm.at[src_v.at[c]], rows_v)
      pltpu.sync_copy(rows_v, acc_sh.at[dst_v.at[c]], add=True)
      return carry

    lax.fori_loop(0, NCH, body, 0)
    plsc.subcore_barrier()
    pltpu.sync_copy(acc_sh.at[pl.ds(sid * RPS, RPS)],
                    out_hbm.at[cid, pl.ds(sid * RPS, RPS)])

  return edge_kernel


def _make_decode_kernel(d):
  @functools.partial(
      pl.kernel, mesh=_MESH,
      compiler_params=pltpu.CompilerParams(use_tc_tiling_on_sc=False, needs_layout_passes=False),
      out_type=jax.ShapeDtypeStruct((NW, NCH, CH), jnp.float32),
      scratch_types=[
          pltpu.VMEM((NCH, CH), jnp.int32),
          pltpu.VMEM((NCH, CH), jnp.int32),
          pltpu.VMEM((CH, d), jnp.float32),
          pltpu.VMEM((CH, d), jnp.float32),
          pltpu.VMEM((NCH, CH), jnp.float32),
      ],
  )
  def decode_kernel(z_hbm, src_hbm, dst_hbm, out_hbm,
                    src_v, dst_v, zr_v, zc_v, out_v):
    cid = lax.axis_index("c")
    sid = lax.axis_index("s")
    wid = cid * NS + sid
    pltpu.sync_copy(src_hbm.at[wid], src_v)
    pltpu.sync_copy(dst_hbm.at[wid], dst_v)

    def body(c, carry):
      pltpu.sync_copy(z_hbm.at[src_v.at[c]], zr_v)
      pltpu.sync_copy(z_hbm.at[dst_v.at[c]], zc_v)
      for g in range(CH // 16):
        rows = lax.iota(jnp.int32, 16) + (g * 16)
        acc = jnp.zeros((16,), jnp.float32)
        for j in range(d):
          col = jnp.full((16,), j, jnp.int32)
          acc = acc + (plsc.load_gather(zr_v, [rows, col]) *
                       plsc.load_gather(zc_v, [rows, col]))
        out_v[c, pl.ds(g * 16, 16)] = acc
      return carry

    lax.fori_loop(0, NCH, body, 0)
    pltpu.sync_copy(out_v, out_hbm.at[wid])

  return decode_kernel


def _tc_y1(x_pad, W1, degp):
  blk = 1024
  grid = N_PAD // blk

  def body(xb, wb, d0b, d1b, yb):
    dinv = lax.rsqrt(d0b[...] + d1b[...] + 1.0)
    yb[...] = dinv * jnp.dot(xb[...], wb[...],
                             preferred_element_type=jnp.float32)

  return pl.pallas_call(
      body,
      grid=(grid,),
      in_specs=[
          pl.BlockSpec((blk, 128), lambda i: (i, 0)),
          pl.BlockSpec((128, 128), lambda i: (0, 0)),
          pl.BlockSpec((blk, 1), lambda i: (i, 0)),
          pl.BlockSpec((blk, 1), lambda i: (i, 0)),
      ],
      out_specs=pl.BlockSpec((blk, 128), lambda i: (i, 0)),
      out_shape=jax.ShapeDtypeStruct((N_PAD, 128), jnp.float32),
  )(x_pad, W1, degp[0], degp[1])


def _tc_layer2_in(S1p, y1, degp, b1, W2):
  blk = 1024
  grid = N_PAD // blk

  def body(sb, yb, d0b, d1b, bb, wb, ob):
    dinv = lax.rsqrt(d0b[...] + d1b[...] + 1.0)
    h = jax.nn.relu(dinv * (sb[0] + sb[1] + yb[...]) + bb[...])
    ob[...] = dinv * jnp.dot(h, wb[...], preferred_element_type=jnp.float32)

  return pl.pallas_call(
      body,
      grid=(grid,),
      in_specs=[
          pl.BlockSpec((2, blk, 128), lambda i: (0, i, 0)),
          pl.BlockSpec((blk, 128), lambda i: (i, 0)),
          pl.BlockSpec((blk, 1), lambda i: (i, 0)),
          pl.BlockSpec((blk, 1), lambda i: (i, 0)),
          pl.BlockSpec((1, 128), lambda i: (0, 0)),
          pl.BlockSpec((128, 64), lambda i: (0, 0)),
      ],
      out_specs=pl.BlockSpec((blk, 64), lambda i: (i, 0)),
      out_shape=jax.ShapeDtypeStruct((N_PAD, 64), jnp.float32),
  )(S1p, y1, degp[0], degp[1], b1, W2)


def _tc_z(S2p, y2, degp, b2):
  blk = 1024
  grid = N_PAD // blk

  def body(sb, yb, d0b, d1b, bb, zb):
    dinv = lax.rsqrt(d0b[...] + d1b[...] + 1.0)
    zb[...] = dinv * (sb[0] + sb[1] + yb[...]) + bb[...]

  return pl.pallas_call(
      body,
      grid=(grid,),
      in_specs=[
          pl.BlockSpec((2, blk, 64), lambda i: (0, i, 0)),
          pl.BlockSpec((blk, 64), lambda i: (i, 0)),
          pl.BlockSpec((blk, 1), lambda i: (i, 0)),
          pl.BlockSpec((blk, 1), lambda i: (i, 0)),
          pl.BlockSpec((1, 64), lambda i: (0, 0)),
      ],
      out_specs=pl.BlockSpec((blk, 64), lambda i: (i, 0)),
      out_shape=jax.ShapeDtypeStruct((N_PAD, 64), jnp.float32),
  )(S2p, y2, degp[0], degp[1], b2)


_deg_kernel = _make_deg_kernel()
_edge_kernel_128 = _make_edge_kernel(128)
_edge_kernel_64 = _make_edge_kernel(64)
_decode_kernel = _make_decode_kernel(64)


def kernel(x, edge_index, W1, b1, W2, b2):
  n, d_in = x.shape
  e = edge_index.shape[1]
  x_pad = jnp.pad(x, ((0, N_PAD - n), (0, 0)))
  src_r = edge_index[0].reshape(NW, NCH, CH)
  dst_r = edge_index[1].reshape(NW, NCH, CH)
  deg_part = _deg_kernel(dst_r, jnp.zeros((RPS, 16), jnp.float32))
  degp = deg_part[:, :, 0:1]

  y1 = _tc_y1(x_pad, W1, degp)
  S1p = _edge_kernel_128(y1, src_r, dst_r, jnp.zeros((RPS, 128), jnp.float32))
  y2 = _tc_layer2_in(S1p, y1, degp, b1.reshape(1, 128), W2)
  S2p = _edge_kernel_64(y2, src_r, dst_r, jnp.zeros((RPS, 64), jnp.float32))
  z = _tc_z(S2p, y2, degp, b2.reshape(1, 64))

  scores = _decode_kernel(z, src_r, dst_r)
  return scores.reshape(e)


# speedup vs baseline: 11.4825x; 1.3211x over previous
import functools

import jax
import jax.numpy as jnp
from jax import lax
from jax.experimental import pallas as pl
from jax.experimental.pallas import tpu as pltpu
from jax.experimental.pallas import tpu_sc as plsc

N_PAD = 10240
NC, NS, NW = 2, 16, 32
CH = 80
NCH = 125
RPS = N_PAD // NS

_MESH = plsc.VectorSubcoreMesh(
    core_axis_name="c", subcore_axis_name="s", num_cores=NC, num_subcores=NS)


def _make_deg_kernel():
  @functools.partial(
      pl.kernel, mesh=_MESH,
      compiler_params=pltpu.CompilerParams(use_tc_tiling_on_sc=False, needs_layout_passes=False),
      out_type=jax.ShapeDtypeStruct((NC, N_PAD, 16), jnp.float32),
      scratch_types=[
          pltpu.VMEM((NCH, CH), jnp.int32),
          pltpu.VMEM((CH, 16), jnp.float32),
          pltpu.VMEM_SHARED((N_PAD, 16), jnp.float32),
      ],
  )
  def deg_kernel(dst_hbm, zeros_hbm, out_hbm, idx_v, ones_v, deg_sh):
    cid = lax.axis_index("c")
    sid = lax.axis_index("s")
    wid = cid * NS + sid
    pltpu.sync_copy(zeros_hbm, deg_sh.at[pl.ds(sid * RPS, RPS)])
    pltpu.sync_copy(dst_hbm.at[wid], idx_v)
    for r in range(CH):
      ones_v[r, :] = jnp.full((16,), 1.0, jnp.float32)
    plsc.subcore_barrier()

    def body(c, carry):
      pltpu.sync_copy(ones_v, deg_sh.at[idx_v.at[c]], add=True)
      return carry

    lax.fori_loop(0, NCH, body, 0)
    plsc.subcore_barrier()
    pltpu.sync_copy(deg_sh.at[pl.ds(sid * RPS, RPS)],
                    out_hbm.at[cid, pl.ds(sid * RPS, RPS)])

  return deg_kernel


def _make_edge_kernel(d):
  @functools.partial(
      pl.kernel, mesh=_MESH,
      compiler_params=pltpu.CompilerParams(use_tc_tiling_on_sc=False, needs_layout_passes=False),
      out_type=jax.ShapeDtypeStruct((NC, N_PAD, d), jnp.float32),
      scratch_types=[
          pltpu.VMEM((NCH, CH), jnp.int32),
          pltpu.VMEM((NCH, CH), jnp.int32),
          pltpu.VMEM((CH, d), jnp.float32),
          pltpu.VMEM((CH, d), jnp.float32),
          pltpu.VMEM_SHARED((N_PAD, d), jnp.float32),
          pltpu.SemaphoreType.DMA,
          pltpu.SemaphoreType.DMA,
      ],
  )
  def edge_kernel(y_hbm, src_hbm, dst_hbm, zeros_hbm, out_hbm,
                  src_v, dst_v, rows0, rows1, acc_sh, sem_g, sem_s):
    cid = lax.axis_index("c")
    sid = lax.axis_index("s")
    wid = cid * NS + sid
    pltpu.sync_copy(zeros_hbm, acc_sh.at[pl.ds(sid * RPS, RPS)])
    pltpu.sync_copy(src_hbm.at[wid], src_v)
    pltpu.sync_copy(dst_hbm.at[wid], dst_v)
    plsc.subcore_barrier()

    pltpu.async_copy(y_hbm.at[src_v.at[0]], rows0, sem_g)

    def half(c, cur, oth):
      @pl.when(c > 0)
      def _():
        pltpu.make_async_copy(oth, acc_sh.at[dst_v.at[c - 1]], sem_s).wait()

      @pl.when(c + 1 < NCH)
      def _():
        pltpu.async_copy(y_hbm.at[src_v.at[c + 1]], oth, sem_g)

      pltpu.make_async_copy(y_hbm.at[src_v.at[c]], cur, sem_g).wait()
      pltpu.async_copy(cur, acc_sh.at[dst_v.at[c]], sem_s, add=True)

    def body(k, carry):
      c = k * 2
      half(c, rows0, rows1)
      half(c + 1, rows1, rows0)
      return carry

    lax.fori_loop(0, NCH // 2, body, 0)
    half(NCH - 1, rows0, rows1)
    pltpu.make_async_copy(rows0, acc_sh.at[dst_v.at[NCH - 1]], sem_s).wait()
    plsc.subcore_barrier()
    pltpu.sync_copy(acc_sh.at[pl.ds(sid * RPS, RPS)],
                    out_hbm.at[cid, pl.ds(sid * RPS, RPS)])

  return edge_kernel


def _make_decode_kernel(d):
  @functools.partial(
      pl.kernel, mesh=_MESH,
      compiler_params=pltpu.CompilerParams(use_tc_tiling_on_sc=False, needs_layout_passes=False),
      out_type=jax.ShapeDtypeStruct((NW, NCH, CH), jnp.float32),
      scratch_types=[
          pltpu.VMEM((NCH, CH), jnp.int32),
          pltpu.VMEM((NCH, CH), jnp.int32),
          pltpu.VMEM((CH, d), jnp.float32),
          pltpu.VMEM((CH, d), jnp.float32),
          pltpu.VMEM((CH, d), jnp.float32),
          pltpu.VMEM((CH, d), jnp.float32),
          pltpu.VMEM((NCH, CH), jnp.float32),
          pltpu.SemaphoreType.DMA,
      ],
  )
  def decode_kernel(z_hbm, src_hbm, dst_hbm, out_hbm,
                    src_v, dst_v, zr0, zc0, zr1, zc1, out_v, sem_g):
    cid = lax.axis_index("c")
    sid = lax.axis_index("s")
    wid = cid * NS + sid
    pltpu.sync_copy(src_hbm.at[wid], src_v)
    pltpu.sync_copy(dst_hbm.at[wid], dst_v)

    pltpu.async_copy(z_hbm.at[src_v.at[0]], zr0, sem_g)
    pltpu.async_copy(z_hbm.at[dst_v.at[0]], zc0, sem_g)

    def half(c, zr_cur, zc_cur, zr_oth, zc_oth):
      @pl.when(c + 1 < NCH)
      def _():
        pltpu.async_copy(z_hbm.at[src_v.at[c + 1]], zr_oth, sem_g)
        pltpu.async_copy(z_hbm.at[dst_v.at[c + 1]], zc_oth, sem_g)

      pltpu.make_async_copy(z_hbm.at[src_v.at[c]], zr_cur, sem_g).wait()
      pltpu.make_async_copy(z_hbm.at[dst_v.at[c]], zc_cur, sem_g).wait()
      for g in range(CH // 16):
        rows = lax.iota(jnp.int32, 16) + (g * 16)
        accs = [jnp.zeros((16,), jnp.float32) for _ in range(4)]
        for j in range(d):
          col = jnp.full((16,), j, jnp.int32)
          accs[j % 4] = accs[j % 4] + (plsc.load_gather(zr_cur, [rows, col]) *
                                       plsc.load_gather(zc_cur, [rows, col]))
        out_v[c, pl.ds(g * 16, 16)] = (accs[0] + accs[1]) + (accs[2] + accs[3])

    def body(k, carry):
      c = k * 2
      half(c, zr0, zc0, zr1, zc1)
      half(c + 1, zr1, zc1, zr0, zc0)
      return carry

    lax.fori_loop(0, NCH // 2, body, 0)
    half(NCH - 1, zr0, zc0, zr1, zc1)
    pltpu.sync_copy(out_v, out_hbm.at[wid])

  return decode_kernel


def _tc_y1(x_pad, W1, degp):
  blk = 1024
  grid = N_PAD // blk

  def body(xb, wb, d0b, d1b, yb):
    dinv = lax.rsqrt(d0b[...] + d1b[...] + 1.0)
    yb[...] = dinv * jnp.dot(xb[...], wb[...],
                             preferred_element_type=jnp.float32)

  return pl.pallas_call(
      body,
      grid=(grid,),
      in_specs=[
          pl.BlockSpec((blk, 128), lambda i: (i, 0)),
          pl.BlockSpec((128, 128), lambda i: (0, 0)),
          pl.BlockSpec((blk, 1), lambda i: (i, 0)),
          pl.BlockSpec((blk, 1), lambda i: (i, 0)),
      ],
      out_specs=pl.BlockSpec((blk, 128), lambda i: (i, 0)),
      out_shape=jax.ShapeDtypeStruct((N_PAD, 128), jnp.float32),
  )(x_pad, W1, degp[0], degp[1])


def _tc_layer2_in(S1p, y1, degp, b1, W2):
  blk = 1024
  grid = N_PAD // blk

  def body(sb, yb, d0b, d1b, bb, wb, ob):
    dinv = lax.rsqrt(d0b[...] + d1b[...] + 1.0)
    h = jax.nn.relu(dinv * (sb[0] + sb[1] + yb[...]) + bb[...])
    ob[...] = dinv * jnp.dot(h, wb[...], preferred_element_type=jnp.float32)

  return pl.pallas_call(
      body,
      grid=(grid,),
      in_specs=[
          pl.BlockSpec((2, blk, 128), lambda i: (0, i, 0)),
          pl.BlockSpec((blk, 128), lambda i: (i, 0)),
          pl.BlockSpec((blk, 1), lambda i: (i, 0)),
          pl.BlockSpec((blk, 1), lambda i: (i, 0)),
          pl.BlockSpec((1, 128), lambda i: (0, 0)),
          pl.BlockSpec((128, 64), lambda i: (0, 0)),
      ],
      out_specs=pl.BlockSpec((blk, 64), lambda i: (i, 0)),
      out_shape=jax.ShapeDtypeStruct((N_PAD, 64), jnp.float32),
  )(S1p, y1, degp[0], degp[1], b1, W2)


def _tc_z(S2p, y2, degp, b2):
  blk = 1024
  grid = N_PAD // blk

  def body(sb, yb, d0b, d1b, bb, zb):
    dinv = lax.rsqrt(d0b[...] + d1b[...] + 1.0)
    zb[...] = dinv * (sb[0] + sb[1] + yb[...]) + bb[...]

  return pl.pallas_call(
      body,
      grid=(grid,),
      in_specs=[
          pl.BlockSpec((2, blk, 64), lambda i: (0, i, 0)),
          pl.BlockSpec((blk, 64), lambda i: (i, 0)),
          pl.BlockSpec((blk, 1), lambda i: (i, 0)),
          pl.BlockSpec((blk, 1), lambda i: (i, 0)),
          pl.BlockSpec((1, 64), lambda i: (0, 0)),
      ],
      out_specs=pl.BlockSpec((blk, 64), lambda i: (i, 0)),
      out_shape=jax.ShapeDtypeStruct((N_PAD, 64), jnp.float32),
  )(S2p, y2, degp[0], degp[1], b2)


_deg_kernel = _make_deg_kernel()
_edge_kernel_128 = _make_edge_kernel(128)
_edge_kernel_64 = _make_edge_kernel(64)
_decode_kernel = _make_decode_kernel(64)


def kernel(x, edge_index, W1, b1, W2, b2):
  n, d_in = x.shape
  e = edge_index.shape[1]
  x_pad = jnp.pad(x, ((0, N_PAD - n), (0, 0)))
  src_r = edge_index[0].reshape(NW, NCH, CH)
  dst_r = edge_index[1].reshape(NW, NCH, CH)
  deg_part = _deg_kernel(dst_r, jnp.zeros((RPS, 16), jnp.float32))
  degp = deg_part[:, :, 0:1]

  y1 = _tc_y1(x_pad, W1, degp)
  S1p = _edge_kernel_128(y1, src_r, dst_r, jnp.zeros((RPS, 128), jnp.float32))
  y2 = _tc_layer2_in(S1p, y1, degp, b1.reshape(1, 128), W2)
  S2p = _edge_kernel_64(y2, src_r, dst_r, jnp.zeros((RPS, 64), jnp.float32))
  z = _tc_z(S2p, y2, degp, b2.reshape(1, 64))

  scores = _decode_kernel(z, src_r, dst_r)
  return scores.reshape(e)


# speedup vs baseline: 19.9436x; 1.7369x over previous
import functools

import jax
import jax.numpy as jnp
from jax import lax
from jax.experimental import pallas as pl
from jax.experimental.pallas import tpu as pltpu
from jax.experimental.pallas import tpu_sc as plsc

N_PAD = 10240
NC, NS, NW = 2, 16, 32
CH = 80
NCH = 125
RPS = N_PAD // NS

_MESH = plsc.VectorSubcoreMesh(
    core_axis_name="c", subcore_axis_name="s", num_cores=NC, num_subcores=NS)


def _make_deg_kernel():
  @functools.partial(
      pl.kernel, mesh=_MESH,
      compiler_params=pltpu.CompilerParams(use_tc_tiling_on_sc=False, needs_layout_passes=False),
      out_type=jax.ShapeDtypeStruct((NC, N_PAD, 16), jnp.float32),
      scratch_types=[
          pltpu.VMEM((NCH, CH), jnp.int32),
          pltpu.VMEM((CH, 16), jnp.float32),
          pltpu.VMEM_SHARED((N_PAD, 16), jnp.float32),
      ],
  )
  def deg_kernel(dst_hbm, zeros_hbm, out_hbm, idx_v, ones_v, deg_sh):
    cid = lax.axis_index("c")
    sid = lax.axis_index("s")
    wid = cid * NS + sid
    pltpu.sync_copy(zeros_hbm, deg_sh.at[pl.ds(sid * RPS, RPS)])
    pltpu.sync_copy(dst_hbm.at[wid], idx_v)
    for r in range(CH):
      ones_v[r, :] = jnp.full((16,), 1.0, jnp.float32)
    plsc.subcore_barrier()

    def body(c, carry):
      pltpu.sync_copy(ones_v, deg_sh.at[idx_v.at[c]], add=True)
      return carry

    lax.fori_loop(0, NCH, body, 0)
    plsc.subcore_barrier()
    pltpu.sync_copy(deg_sh.at[pl.ds(sid * RPS, RPS)],
                    out_hbm.at[cid, pl.ds(sid * RPS, RPS)])

  return deg_kernel


def _make_edge_kernel(d):
  @functools.partial(
      pl.kernel, mesh=_MESH,
      compiler_params=pltpu.CompilerParams(use_tc_tiling_on_sc=False, needs_layout_passes=False),
      out_type=jax.ShapeDtypeStruct((NC, N_PAD, d), jnp.float32),
      scratch_types=[
          pltpu.VMEM((NCH, CH), jnp.int32),
          pltpu.VMEM((NCH, CH), jnp.int32),
          pltpu.VMEM((CH, d), jnp.float32),
          pltpu.VMEM((CH, d), jnp.float32),
          pltpu.VMEM_SHARED((N_PAD, d), jnp.float32),
          pltpu.SemaphoreType.DMA,
          pltpu.SemaphoreType.DMA,
      ],
  )
  def edge_kernel(y_hbm, src_hbm, dst_hbm, zeros_hbm, out_hbm,
                  src_v, dst_v, rows0, rows1, acc_sh, sem_g, sem_s):
    cid = lax.axis_index("c")
    sid = lax.axis_index("s")
    wid = cid * NS + sid
    pltpu.sync_copy(zeros_hbm, acc_sh.at[pl.ds(sid * RPS, RPS)])
    pltpu.sync_copy(src_hbm.at[wid], src_v)
    pltpu.sync_copy(dst_hbm.at[wid], dst_v)
    plsc.subcore_barrier()

    pltpu.async_copy(y_hbm.at[src_v.at[0]], rows0, sem_g)

    def half(c, cur, oth):
      @pl.when(c > 0)
      def _():
        pltpu.make_async_copy(oth, acc_sh.at[dst_v.at[c - 1]], sem_s).wait()

      @pl.when(c + 1 < NCH)
      def _():
        pltpu.async_copy(y_hbm.at[src_v.at[c + 1]], oth, sem_g)

      pltpu.make_async_copy(y_hbm.at[src_v.at[c]], cur, sem_g).wait()
      pltpu.async_copy(cur, acc_sh.at[dst_v.at[c]], sem_s, add=True)

    def body(k, carry):
      c = k * 2
      half(c, rows0, rows1)
      half(c + 1, rows1, rows0)
      return carry

    lax.fori_loop(0, NCH // 2, body, 0)
    half(NCH - 1, rows0, rows1)
    pltpu.make_async_copy(rows0, acc_sh.at[dst_v.at[NCH - 1]], sem_s).wait()
    plsc.subcore_barrier()
    pltpu.sync_copy(acc_sh.at[pl.ds(sid * RPS, RPS)],
                    out_hbm.at[cid, pl.ds(sid * RPS, RPS)])

  return edge_kernel


def _make_decode_kernel(d):
  @functools.partial(
      pl.kernel, mesh=_MESH,
      compiler_params=pltpu.CompilerParams(use_tc_tiling_on_sc=False, needs_layout_passes=False),
      out_type=jax.ShapeDtypeStruct((NW, NCH, CH), jnp.float32),
      scratch_types=[
          pltpu.VMEM((NCH, CH), jnp.int32),
          pltpu.VMEM((NCH, CH), jnp.int32),
          pltpu.VMEM((CH, d), jnp.float32),
          pltpu.VMEM((CH, d), jnp.float32),
          pltpu.VMEM((CH, d), jnp.float32),
          pltpu.VMEM((CH, d), jnp.float32),
          pltpu.VMEM((NCH, CH), jnp.float32),
          pltpu.SemaphoreType.DMA,
      ],
  )
  def decode_kernel(z_hbm, src_hbm, dst_hbm, out_hbm,
                    src_v, dst_v, zr0, zc0, zr1, zc1, out_v, sem_g):
    cid = lax.axis_index("c")
    sid = lax.axis_index("s")
    wid = cid * NS + sid
    pltpu.sync_copy(src_hbm.at[wid], src_v)
    pltpu.sync_copy(dst_hbm.at[wid], dst_v)

    pltpu.async_copy(z_hbm.at[src_v.at[0]], zr0, sem_g)
    pltpu.async_copy(z_hbm.at[dst_v.at[0]], zc0, sem_g)

    def half(c, zr_cur, zc_cur, zr_oth, zc_oth):
      @pl.when(c + 1 < NCH)
      def _():
        pltpu.async_copy(z_hbm.at[src_v.at[c + 1]], zr_oth, sem_g)
        pltpu.async_copy(z_hbm.at[dst_v.at[c + 1]], zc_oth, sem_g)

      pltpu.make_async_copy(z_hbm.at[src_v.at[c]], zr_cur, sem_g).wait()
      pltpu.make_async_copy(z_hbm.at[dst_v.at[c]], zc_cur, sem_g).wait()
      lanes = lax.iota(jnp.int32, 16)
      cols = [((lanes + t) & 15) + (jb * 16)
              for jb in range(d // 16) for t in range(16)]
      for g in range(CH // 16):
        rows = lanes + (g * 16)
        accs = [jnp.zeros((16,), jnp.float32) for _ in range(4)]
        for i, col in enumerate(cols):
          accs[i % 4] = accs[i % 4] + (plsc.load_gather(zr_cur, [rows, col]) *
                                       plsc.load_gather(zc_cur, [rows, col]))
        out_v[c, pl.ds(g * 16, 16)] = (accs[0] + accs[1]) + (accs[2] + accs[3])

    def body(k, carry):
      c = k * 2
      half(c, zr0, zc0, zr1, zc1)
      half(c + 1, zr1, zc1, zr0, zc0)
      return carry

    lax.fori_loop(0, NCH // 2, body, 0)
    half(NCH - 1, zr0, zc0, zr1, zc1)
    pltpu.sync_copy(out_v, out_hbm.at[wid])

  return decode_kernel


def _tc_y1(x_pad, W1, degp):
  blk = 1024
  grid = N_PAD // blk

  def body(xb, wb, d0b, d1b, yb):
    dinv = lax.rsqrt(d0b[...] + d1b[...] + 1.0)
    yb[...] = dinv * jnp.dot(xb[...], wb[...],
                             preferred_element_type=jnp.float32)

  return pl.pallas_call(
      body,
      grid=(grid,),
      in_specs=[
          pl.BlockSpec((blk, 128), lambda i: (i, 0)),
          pl.BlockSpec((128, 128), lambda i: (0, 0)),
          pl.BlockSpec((blk, 1), lambda i: (i, 0)),
          pl.BlockSpec((blk, 1), lambda i: (i, 0)),
      ],
      out_specs=pl.BlockSpec((blk, 128), lambda i: (i, 0)),
      out_shape=jax.ShapeDtypeStruct((N_PAD, 128), jnp.float32),
  )(x_pad, W1, degp[0], degp[1])


def _tc_layer2_in(S1p, y1, degp, b1, W2):
  blk = 1024
  grid = N_PAD // blk

  def body(sb, yb, d0b, d1b, bb, wb, ob):
    dinv = lax.rsqrt(d0b[...] + d1b[...] + 1.0)
    h = jax.nn.relu(dinv * (sb[0] + sb[1] + yb[...]) + bb[...])
    ob[...] = dinv * jnp.dot(h, wb[...], preferred_element_type=jnp.float32)

  return pl.pallas_call(
      body,
      grid=(grid,),
      in_specs=[
          pl.BlockSpec((2, blk, 128), lambda i: (0, i, 0)),
          pl.BlockSpec((blk, 128), lambda i: (i, 0)),
          pl.BlockSpec((blk, 1), lambda i: (i, 0)),
          pl.BlockSpec((blk, 1), lambda i: (i, 0)),
          pl.BlockSpec((1, 128), lambda i: (0, 0)),
          pl.BlockSpec((128, 64), lambda i: (0, 0)),
      ],
      out_specs=pl.BlockSpec((blk, 64), lambda i: (i, 0)),
      out_shape=jax.ShapeDtypeStruct((N_PAD, 64), jnp.float32),
  )(S1p, y1, degp[0], degp[1], b1, W2)


def _tc_z(S2p, y2, degp, b2):
  blk = 1024
  grid = N_PAD // blk

  def body(sb, yb, d0b, d1b, bb, zb):
    dinv = lax.rsqrt(d0b[...] + d1b[...] + 1.0)
    zb[...] = dinv * (sb[0] + sb[1] + yb[...]) + bb[...]

  return pl.pallas_call(
      body,
      grid=(grid,),
      in_specs=[
          pl.BlockSpec((2, blk, 64), lambda i: (0, i, 0)),
          pl.BlockSpec((blk, 64), lambda i: (i, 0)),
          pl.BlockSpec((blk, 1), lambda i: (i, 0)),
          pl.BlockSpec((blk, 1), lambda i: (i, 0)),
          pl.BlockSpec((1, 64), lambda i: (0, 0)),
      ],
      out_specs=pl.BlockSpec((blk, 64), lambda i: (i, 0)),
      out_shape=jax.ShapeDtypeStruct((N_PAD, 64), jnp.float32),
  )(S2p, y2, degp[0], degp[1], b2)


_deg_kernel = _make_deg_kernel()
_edge_kernel_128 = _make_edge_kernel(128)
_edge_kernel_64 = _make_edge_kernel(64)
_decode_kernel = _make_decode_kernel(64)


def kernel(x, edge_index, W1, b1, W2, b2):
  n, d_in = x.shape
  e = edge_index.shape[1]
  x_pad = jnp.pad(x, ((0, N_PAD - n), (0, 0)))
  src_r = edge_index[0].reshape(NW, NCH, CH)
  dst_r = edge_index[1].reshape(NW, NCH, CH)
  deg_part = _deg_kernel(dst_r, jnp.zeros((RPS, 16), jnp.float32))
  degp = deg_part[:, :, 0:1]

  y1 = _tc_y1(x_pad, W1, degp)
  S1p = _edge_kernel_128(y1, src_r, dst_r, jnp.zeros((RPS, 128), jnp.float32))
  y2 = _tc_layer2_in(S1p, y1, degp, b1.reshape(1, 128), W2)
  S2p = _edge_kernel_64(y2, src_r, dst_r, jnp.zeros((RPS, 64), jnp.float32))
  z = _tc_z(S2p, y2, degp, b2.reshape(1, 64))

  scores = _decode_kernel(z, src_r, dst_r)
  return scores.reshape(e)


# speedup vs baseline: 22.0237x; 1.1043x over previous
import functools

import jax
import jax.numpy as jnp
from jax import lax
from jax.experimental import pallas as pl
from jax.experimental.pallas import tpu as pltpu
from jax.experimental.pallas import tpu_sc as plsc

N = 10000
NC, NS, NW = 2, 16, 32
RPS = N // NS
EPW = 10000

_MESH = plsc.VectorSubcoreMesh(
    core_axis_name="c", subcore_axis_name="s", num_cores=NC, num_subcores=NS)

_SC_PARAMS = pltpu.CompilerParams(
    use_tc_tiling_on_sc=False, needs_layout_passes=False)


def _make_deg_kernel(ch):
  nch = EPW // ch

  @functools.partial(
      pl.kernel, mesh=_MESH,
      compiler_params=_SC_PARAMS,
      out_type=jax.ShapeDtypeStruct((NC, N, 16), jnp.float32),
      scratch_types=[
          pltpu.VMEM((nch, ch), jnp.int32),
          pltpu.VMEM((ch, 16), jnp.float32),
          pltpu.VMEM_SHARED((N, 16), jnp.float32),
      ],
  )
  def deg_kernel(dst_hbm, zeros_hbm, out_hbm, idx_v, ones_v, deg_sh):
    cid = lax.axis_index("c")
    sid = lax.axis_index("s")
    wid = cid * NS + sid
    pltpu.sync_copy(zeros_hbm, deg_sh.at[pl.ds(sid * RPS, RPS)])
    pltpu.sync_copy(dst_hbm.at[wid], idx_v)
    for r in range(ch):
      ones_v[r, :] = jnp.full((16,), 1.0, jnp.float32)
    plsc.subcore_barrier()

    def body(c, carry):
      pltpu.sync_copy(ones_v, deg_sh.at[idx_v.at[c]], add=True)
      return carry

    lax.fori_loop(0, nch, body, 0)
    plsc.subcore_barrier()
    pltpu.sync_copy(deg_sh.at[pl.ds(sid * RPS, RPS)],
                    out_hbm.at[cid, pl.ds(sid * RPS, RPS)])

  return deg_kernel


def _make_edge_kernel(d, ch):
  nch = EPW // ch

  @functools.partial(
      pl.kernel, mesh=_MESH,
      compiler_params=_SC_PARAMS,
      out_type=jax.ShapeDtypeStruct((NC, N, d), jnp.float32),
      scratch_types=[
          pltpu.VMEM((nch, ch), jnp.int32),
          pltpu.VMEM((nch, ch), jnp.int32),
          pltpu.VMEM((ch, d), jnp.float32),
          pltpu.VMEM((ch, d), jnp.float32),
          pltpu.VMEM((ch, d), jnp.float32),
          pltpu.VMEM((ch, d), jnp.float32),
          pltpu.VMEM_SHARED((N, d), jnp.float32),
          pltpu.SemaphoreType.DMA,
          pltpu.SemaphoreType.DMA,
      ],
  )
  def edge_kernel(y_hbm, src_hbm, dst_hbm, zeros_hbm, out_hbm,
                  src_v, dst_v, rows0, rows1, rows2, rows3, acc_sh,
                  sem_g, sem_s):
    cid = lax.axis_index("c")
    sid = lax.axis_index("s")
    wid = cid * NS + sid
    pltpu.sync_copy(zeros_hbm, acc_sh.at[pl.ds(sid * RPS, RPS)])
    pltpu.sync_copy(src_hbm.at[wid], src_v)
    pltpu.sync_copy(dst_hbm.at[wid], dst_v)
    plsc.subcore_barrier()

    bufs = (rows0, rows1, rows2, rows3)
    pltpu.async_copy(y_hbm.at[src_v.at[0]], bufs[0], sem_g)
    pltpu.async_copy(y_hbm.at[src_v.at[1]], bufs[1], sem_g)

    def step(c, b):
      @pl.when(c >= 2)
      def _():
        pltpu.make_async_copy(bufs[(b + 2) % 4],
                              acc_sh.at[dst_v.at[c - 2]], sem_s).wait()

      @pl.when(c + 2 < nch)
      def _():
        pltpu.async_copy(y_hbm.at[src_v.at[c + 2]], bufs[(b + 2) % 4], sem_g)

      pltpu.make_async_copy(y_hbm.at[src_v.at[c]], bufs[b], sem_g).wait()
      pltpu.async_copy(bufs[b], acc_sh.at[dst_v.at[c]], sem_s, add=True)

    def body(k, carry):
      c = k * 4
      for off in range(4):
        step(c + off, off)
      return carry

    lax.fori_loop(0, nch // 4, body, 0)
    for c in range((nch // 4) * 4, nch):
      step(c, c % 4)
    pltpu.make_async_copy(bufs[(nch - 2) % 4],
                          acc_sh.at[dst_v.at[nch - 2]], sem_s).wait()
    pltpu.make_async_copy(bufs[(nch - 1) % 4],
                          acc_sh.at[dst_v.at[nch - 1]], sem_s).wait()
    plsc.subcore_barrier()
    pltpu.sync_copy(acc_sh.at[pl.ds(sid * RPS, RPS)],
                    out_hbm.at[cid, pl.ds(sid * RPS, RPS)])

  return edge_kernel


def _make_decode_kernel(d, ch):
  nch = EPW // ch

  @functools.partial(
      pl.kernel, mesh=_MESH,
      compiler_params=_SC_PARAMS,
      out_type=jax.ShapeDtypeStruct((NW, nch, ch), jnp.float32),
      scratch_types=[
          pltpu.VMEM((nch, ch), jnp.int32),
          pltpu.VMEM((nch, ch), jnp.int32),
          pltpu.VMEM((ch, d), jnp.float32),
          pltpu.VMEM((ch, d), jnp.float32),
          pltpu.VMEM((ch, d), jnp.float32),
          pltpu.VMEM((ch, d), jnp.float32),
          pltpu.VMEM((nch, ch), jnp.float32),
          pltpu.SemaphoreType.DMA,
      ],
  )
  def decode_kernel(z_hbm, src_hbm, dst_hbm, out_hbm,
                    src_v, dst_v, zr0, zc0, zr1, zc1, out_v, sem_g):
    cid = lax.axis_index("c")
    sid = lax.axis_index("s")
    wid = cid * NS + sid
    pltpu.sync_copy(src_hbm.at[wid], src_v)
    pltpu.sync_copy(dst_hbm.at[wid], dst_v)

    pltpu.async_copy(z_hbm.at[src_v.at[0]], zr0, sem_g)
    pltpu.async_copy(z_hbm.at[dst_v.at[0]], zc0, sem_g)

    def half(c, zr_cur, zc_cur, zr_oth, zc_oth):
      @pl.when(c + 1 < nch)
      def _():
        pltpu.async_copy(z_hbm.at[src_v.at[c + 1]], zr_oth, sem_g)
        pltpu.async_copy(z_hbm.at[dst_v.at[c + 1]], zc_oth, sem_g)

      pltpu.make_async_copy(z_hbm.at[src_v.at[c]], zr_cur, sem_g).wait()
      pltpu.make_async_copy(z_hbm.at[dst_v.at[c]], zc_cur, sem_g).wait()
      lanes = lax.iota(jnp.int32, 16)
      for g in range(ch // 16):
        rows = lanes + (g * 16)
        accs = [jnp.zeros((16,), jnp.float32) for _ in range(4)]
        for t in range(16):
          colt = (lanes + t) & 15
          for jb in range(d // 16):
            col = colt + (jb * 16)
            accs[jb] = accs[jb] + (plsc.load_gather(zr_cur, [rows, col]) *
                                   plsc.load_gather(zc_cur, [rows, col]))
        out_v[c, pl.ds(g * 16, 16)] = (accs[0] + accs[1]) + (accs[2] + accs[3])

    def body(k, carry):
      c = k * 2
      half(c, zr0, zc0, zr1, zc1)
      half(c + 1, zr1, zc1, zr0, zc0)
      return carry

    lax.fori_loop(0, nch // 2, body, 0)
    half(nch - 1, zr0, zc0, zr1, zc1)
    pltpu.sync_copy(out_v, out_hbm.at[wid])

  return decode_kernel


def _tc_y1(x, W1, deg0, deg1):
  blk = 1000
  grid = N // blk

  def body(xb, wb, d0b, d1b, yb):
    dinv = lax.rsqrt(d0b[...] + d1b[...] + 1.0)
    yb[...] = dinv * jnp.dot(xb[...], wb[...],
                             preferred_element_type=jnp.float32)

  return pl.pallas_call(
      body,
      grid=(grid,),
      in_specs=[
          pl.BlockSpec((blk, 128), lambda i: (i, 0)),
          pl.BlockSpec((128, 128), lambda i: (0, 0)),
          pl.BlockSpec((blk, 1), lambda i: (i, 0)),
          pl.BlockSpec((blk, 1), lambda i: (i, 0)),
      ],
      out_specs=pl.BlockSpec((blk, 128), lambda i: (i, 0)),
      out_shape=jax.ShapeDtypeStruct((N, 128), jnp.float32),
  )(x, W1, deg0, deg1)


def _tc_layer2_in(S1p, y1, deg0, deg1, b1, W2):
  blk = 1000
  grid = N // blk

  def body(sb, yb, d0b, d1b, bb, wb, ob):
    dinv = lax.rsqrt(d0b[...] + d1b[...] + 1.0)
    h = jax.nn.relu(dinv * (sb[0] + sb[1] + yb[...]) + bb[...])
    ob[...] = dinv * jnp.dot(h, wb[...], preferred_element_type=jnp.float32)

  return pl.pallas_call(
      body,
      grid=(grid,),
      in_specs=[
          pl.BlockSpec((2, blk, 128), lambda i: (0, i, 0)),
          pl.BlockSpec((blk, 128), lambda i: (i, 0)),
          pl.BlockSpec((blk, 1), lambda i: (i, 0)),
          pl.BlockSpec((blk, 1), lambda i: (i, 0)),
          pl.BlockSpec((1, 128), lambda i: (0, 0)),
          pl.BlockSpec((128, 64), lambda i: (0, 0)),
      ],
      out_specs=pl.BlockSpec((blk, 64), lambda i: (i, 0)),
      out_shape=jax.ShapeDtypeStruct((N, 64), jnp.float32),
  )(S1p, y1, deg0, deg1, b1, W2)


def _tc_z(S2p, y2, deg0, deg1, b2):
  blk = 1000
  grid = N // blk

  def body(sb, yb, d0b, d1b, bb, zb):
    dinv = lax.rsqrt(d0b[...] + d1b[...] + 1.0)
    zb[...] = dinv * (sb[0] + sb[1] + yb[...]) + bb[...]

  return pl.pallas_call(
      body,
      grid=(grid,),
      in_specs=[
          pl.BlockSpec((2, blk, 64), lambda i: (0, i, 0)),
          pl.BlockSpec((blk, 64), lambda i: (i, 0)),
          pl.BlockSpec((blk, 1), lambda i: (i, 0)),
          pl.BlockSpec((blk, 1), lambda i: (i, 0)),
          pl.BlockSpec((1, 64), lambda i: (0, 0)),
      ],
      out_specs=pl.BlockSpec((blk, 64), lambda i: (i, 0)),
      out_shape=jax.ShapeDtypeStruct((N, 64), jnp.float32),
  )(S2p, y2, deg0, deg1, b2)


_deg_kernel = _make_deg_kernel(80)
_edge_kernel_128 = _make_edge_kernel(128, 40)
_edge_kernel_64 = _make_edge_kernel(64, 80)
_decode_kernel = _make_decode_kernel(64, 80)


def kernel(x, edge_index, W1, b1, W2, b2):
  e = edge_index.shape[1]
  src80 = edge_index[0].reshape(NW, EPW // 80, 80)
  dst80 = edge_index[1].reshape(NW, EPW // 80, 80)
  src40 = edge_index[0].reshape(NW, EPW // 40, 40)
  dst40 = edge_index[1].reshape(NW, EPW // 40, 40)

  deg_part = _deg_kernel(dst80, jnp.zeros((RPS, 16), jnp.float32))
  deg0 = deg_part[0, :, 0:1]
  deg1 = deg_part[1, :, 0:1]

  y1 = _tc_y1(x, W1, deg0, deg1)
  S1p = _edge_kernel_128(y1, src40, dst40, jnp.zeros((RPS, 128), jnp.float32))
  y2 = _tc_layer2_in(S1p, y1, deg0, deg1, b1.reshape(1, 128), W2)
  S2p = _edge_kernel_64(y2, src80, dst80, jnp.zeros((RPS, 64), jnp.float32))
  z = _tc_z(S2p, y2, deg0, deg1, b2.reshape(1, 64))

  scores = _decode_kernel(z, src80, dst80)
  return scores.reshape(e)


# speedup vs baseline: 22.0979x; 1.0034x over previous
import functools

import jax
import jax.numpy as jnp
from jax import lax
from jax.experimental import pallas as pl
from jax.experimental.pallas import tpu as pltpu
from jax.experimental.pallas import tpu_sc as plsc

N = 10000
NC, NS, NW = 2, 16, 32
RPS = N // NS
EPW = 10000

_MESH = plsc.VectorSubcoreMesh(
    core_axis_name="c", subcore_axis_name="s", num_cores=NC, num_subcores=NS)

_SC_PARAMS = pltpu.CompilerParams(
    use_tc_tiling_on_sc=False, needs_layout_passes=False,
    disable_bounds_checks=True)


def _make_deg_kernel(ch):
  nch = EPW // ch

  @functools.partial(
      pl.kernel, mesh=_MESH,
      compiler_params=_SC_PARAMS,
      out_type=jax.ShapeDtypeStruct((NC, N, 16), jnp.float32),
      scratch_types=[
          pltpu.VMEM((nch, ch), jnp.int32),
          pltpu.VMEM((ch, 16), jnp.float32),
          pltpu.VMEM_SHARED((N, 16), jnp.float32),
      ],
  )
  def deg_kernel(dst_hbm, zeros_hbm, out_hbm, idx_v, ones_v, deg_sh):
    cid = lax.axis_index("c")
    sid = lax.axis_index("s")
    wid = cid * NS + sid
    pltpu.sync_copy(zeros_hbm, deg_sh.at[pl.ds(sid * RPS, RPS)])
    pltpu.sync_copy(dst_hbm.at[wid], idx_v)
    for r in range(ch):
      ones_v[r, :] = jnp.full((16,), 1.0, jnp.float32)
    plsc.subcore_barrier()

    def body(c, carry):
      pltpu.sync_copy(ones_v, deg_sh.at[idx_v.at[c]], add=True)
      return carry

    lax.fori_loop(0, nch, body, 0)
    plsc.subcore_barrier()
    pltpu.sync_copy(deg_sh.at[pl.ds(sid * RPS, RPS)],
                    out_hbm.at[cid, pl.ds(sid * RPS, RPS)])

  return deg_kernel


def _make_edge_kernel(d, ch):
  nch = EPW // ch

  @functools.partial(
      pl.kernel, mesh=_MESH,
      compiler_params=_SC_PARAMS,
      out_type=jax.ShapeDtypeStruct((NC, N, d), jnp.float32),
      scratch_types=[
          pltpu.VMEM((nch, ch), jnp.int32),
          pltpu.VMEM((nch, ch), jnp.int32),
          pltpu.VMEM((ch, d), jnp.float32),
          pltpu.VMEM((ch, d), jnp.float32),
          pltpu.VMEM((ch, d), jnp.float32),
          pltpu.VMEM((ch, d), jnp.float32),
          pltpu.VMEM_SHARED((N, d), jnp.float32),
          pltpu.SemaphoreType.DMA,
          pltpu.SemaphoreType.DMA,
      ],
  )
  def edge_kernel(y_hbm, src_hbm, dst_hbm, zeros_hbm, out_hbm,
                  src_v, dst_v, rows0, rows1, rows2, rows3, acc_sh,
                  sem_g, sem_s):
    cid = lax.axis_index("c")
    sid = lax.axis_index("s")
    wid = cid * NS + sid
    pltpu.sync_copy(zeros_hbm, acc_sh.at[pl.ds(sid * RPS, RPS)])
    pltpu.sync_copy(src_hbm.at[wid], src_v)
    pltpu.sync_copy(dst_hbm.at[wid], dst_v)
    plsc.subcore_barrier()

    bufs = (rows0, rows1, rows2, rows3)
    pltpu.async_copy(y_hbm.at[src_v.at[0]], bufs[0], sem_g)
    pltpu.async_copy(y_hbm.at[src_v.at[1]], bufs[1], sem_g)

    def step(c, b):
      @pl.when(c >= 2)
      def _():
        pltpu.make_async_copy(bufs[(b + 2) % 4],
                              acc_sh.at[dst_v.at[c - 2]], sem_s).wait()

      @pl.when(c + 2 < nch)
      def _():
        pltpu.async_copy(y_hbm.at[src_v.at[c + 2]], bufs[(b + 2) % 4], sem_g)

      pltpu.make_async_copy(y_hbm.at[src_v.at[c]], bufs[b], sem_g).wait()
      pltpu.async_copy(bufs[b], acc_sh.at[dst_v.at[c]], sem_s, add=True)

    def body(k, carry):
      c = k * 4
      for off in range(4):
        step(c + off, off)
      return carry

    lax.fori_loop(0, nch // 4, body, 0)
    for c in range((nch // 4) * 4, nch):
      step(c, c % 4)
    pltpu.make_async_copy(bufs[(nch - 2) % 4],
                          acc_sh.at[dst_v.at[nch - 2]], sem_s).wait()
    pltpu.make_async_copy(bufs[(nch - 1) % 4],
                          acc_sh.at[dst_v.at[nch - 1]], sem_s).wait()
    plsc.subcore_barrier()
    pltpu.sync_copy(acc_sh.at[pl.ds(sid * RPS, RPS)],
                    out_hbm.at[cid, pl.ds(sid * RPS, RPS)])

  return edge_kernel


def _make_decode_kernel(d, ch):
  nch = EPW // ch

  @functools.partial(
      pl.kernel, mesh=_MESH,
      compiler_params=_SC_PARAMS,
      out_type=jax.ShapeDtypeStruct((NW, nch, ch), jnp.float32),
      scratch_types=[
          pltpu.VMEM((nch, ch), jnp.int32),
          pltpu.VMEM((nch, ch), jnp.int32),
          pltpu.VMEM((ch, d), jnp.float32),
          pltpu.VMEM((ch, d), jnp.float32),
          pltpu.VMEM((ch, d), jnp.float32),
          pltpu.VMEM((ch, d), jnp.float32),
          pltpu.VMEM((nch, ch), jnp.float32),
          pltpu.SemaphoreType.DMA,
      ],
  )
  def decode_kernel(z_hbm, src_hbm, dst_hbm, out_hbm,
                    src_v, dst_v, zr0, zc0, zr1, zc1, out_v, sem_g):
    cid = lax.axis_index("c")
    sid = lax.axis_index("s")
    wid = cid * NS + sid
    pltpu.sync_copy(src_hbm.at[wid], src_v)
    pltpu.sync_copy(dst_hbm.at[wid], dst_v)

    pltpu.async_copy(z_hbm.at[src_v.at[0]], zr0, sem_g)
    pltpu.async_copy(z_hbm.at[dst_v.at[0]], zc0, sem_g)

    lanes = lax.iota(jnp.int32, 16)

    def half(c, zr_cur, zc_cur, zr_oth, zc_oth):
      @pl.when(c + 1 < nch)
      def _():
        pltpu.async_copy(z_hbm.at[src_v.at[c + 1]], zr_oth, sem_g)
        pltpu.async_copy(z_hbm.at[dst_v.at[c + 1]], zc_oth, sem_g)

      pltpu.make_async_copy(z_hbm.at[src_v.at[c]], zr_cur, sem_g).wait()
      pltpu.make_async_copy(z_hbm.at[dst_v.at[c]], zc_cur, sem_g).wait()
      for g in range(ch // 16):
        rows = lanes + (g * 16)
        accs = [jnp.zeros((16,), jnp.float32) for _ in range(4)]
        for t in range(16):
          colt = (lanes + t) & 15
          for jb in range(d // 16):
            col = colt + (jb * 16)
            accs[jb] = accs[jb] + (plsc.load_gather(zr_cur, [rows, col]) *
                                   plsc.load_gather(zc_cur, [rows, col]))
        out_v[c, pl.ds(g * 16, 16)] = (accs[0] + accs[1]) + (accs[2] + accs[3])

    def body(k, carry):
      c = k * 2
      half(c, zr0, zc0, zr1, zc1)
      half(c + 1, zr1, zc1, zr0, zc0)
      return carry

    lax.fori_loop(0, nch // 2, body, 0)
    half(nch - 1, zr0, zc0, zr1, zc1)
    pltpu.sync_copy(out_v, out_hbm.at[wid])

  return decode_kernel


def _tc_y1(x, W1, deg0, deg1):
  blk = 1000
  grid = N // blk

  def body(xb, wb, d0b, d1b, yb):
    dinv = lax.rsqrt(d0b[...] + d1b[...] + 1.0)
    yb[...] = dinv * jnp.dot(xb[...], wb[...],
                             preferred_element_type=jnp.float32)

  return pl.pallas_call(
      body,
      grid=(grid,),
      in_specs=[
          pl.BlockSpec((blk, 128), lambda i: (i, 0)),
          pl.BlockSpec((128, 128), lambda i: (0, 0)),
          pl.BlockSpec((blk, 1), lambda i: (i, 0)),
          pl.BlockSpec((blk, 1), lambda i: (i, 0)),
      ],
      out_specs=pl.BlockSpec((blk, 128), lambda i: (i, 0)),
      out_shape=jax.ShapeDtypeStruct((N, 128), jnp.float32),
  )(x, W1, deg0, deg1)


def _tc_layer2_in(S1p, y1, deg0, deg1, b1, W2):
  blk = 1000
  grid = N // blk

  def body(sb, yb, d0b, d1b, bb, wb, ob):
    dinv = lax.rsqrt(d0b[...] + d1b[...] + 1.0)
    h = jax.nn.relu(dinv * (sb[0] + sb[1] + yb[...]) + bb[...])
    ob[...] = dinv * jnp.dot(h, wb[...], preferred_element_type=jnp.float32)

  return pl.pallas_call(
      body,
      grid=(grid,),
      in_specs=[
          pl.BlockSpec((2, blk, 128), lambda i: (0, i, 0)),
          pl.BlockSpec((blk, 128), lambda i: (i, 0)),
          pl.BlockSpec((blk, 1), lambda i: (i, 0)),
          pl.BlockSpec((blk, 1), lambda i: (i, 0)),
          pl.BlockSpec((1, 128), lambda i: (0, 0)),
          pl.BlockSpec((128, 64), lambda i: (0, 0)),
      ],
      out_specs=pl.BlockSpec((blk, 64), lambda i: (i, 0)),
      out_shape=jax.ShapeDtypeStruct((N, 64), jnp.float32),
  )(S1p, y1, deg0, deg1, b1, W2)


def _tc_z(S2p, y2, deg0, deg1, b2):
  blk = 1000
  grid = N // blk

  def body(sb, yb, d0b, d1b, bb, zb):
    dinv = lax.rsqrt(d0b[...] + d1b[...] + 1.0)
    zb[...] = dinv * (sb[0] + sb[1] + yb[...]) + bb[...]

  return pl.pallas_call(
      body,
      grid=(grid,),
      in_specs=[
          pl.BlockSpec((2, blk, 64), lambda i: (0, i, 0)),
          pl.BlockSpec((blk, 64), lambda i: (i, 0)),
          pl.BlockSpec((blk, 1), lambda i: (i, 0)),
          pl.BlockSpec((blk, 1), lambda i: (i, 0)),
          pl.BlockSpec((1, 64), lambda i: (0, 0)),
      ],
      out_specs=pl.BlockSpec((blk, 64), lambda i: (i, 0)),
      out_shape=jax.ShapeDtypeStruct((N, 64), jnp.float32),
  )(S2p, y2, deg0, deg1, b2)


_deg_kernel = _make_deg_kernel(80)
_edge_kernel_128 = _make_edge_kernel(128, 40)
_edge_kernel_64 = _make_edge_kernel(64, 80)
_decode_kernel = _make_decode_kernel(64, 80)


def kernel(x, edge_index, W1, b1, W2, b2):
  e = edge_index.shape[1]
  src80 = edge_index[0].reshape(NW, EPW // 80, 80)
  dst80 = edge_index[1].reshape(NW, EPW // 80, 80)
  src40 = edge_index[0].reshape(NW, EPW // 40, 40)
  dst40 = edge_index[1].reshape(NW, EPW // 40, 40)

  deg_part = _deg_kernel(dst80, jnp.zeros((RPS, 16), jnp.float32))
  deg0 = deg_part[0, :, 0:1]
  deg1 = deg_part[1, :, 0:1]

  y1 = _tc_y1(x, W1, deg0, deg1)
  S1p = _edge_kernel_128(y1, src40, dst40, jnp.zeros((RPS, 128), jnp.float32))
  y2 = _tc_layer2_in(S1p, y1, deg0, deg1, b1.reshape(1, 128), W2)
  S2p = _edge_kernel_64(y2, src80, dst80, jnp.zeros((RPS, 64), jnp.float32))
  z = _tc_z(S2p, y2, deg0, deg1, b2.reshape(1, 64))

  scores = _decode_kernel(z, src80, dst80)
  return scores.reshape(e)


# speedup vs baseline: 22.5922x; 1.0224x over previous
import functools

import jax
import jax.numpy as jnp
from jax import lax
from jax.experimental import pallas as pl
from jax.experimental.pallas import tpu as pltpu
from jax.experimental.pallas import tpu_sc as plsc

N = 10000
NC, NS, NW = 2, 16, 32
RPS = N // NS
EPW = 10000

_MESH = plsc.VectorSubcoreMesh(
    core_axis_name="c", subcore_axis_name="s", num_cores=NC, num_subcores=NS)

_SC_PARAMS = pltpu.CompilerParams(
    use_tc_tiling_on_sc=False, needs_layout_passes=False,
    disable_bounds_checks=True)


def _make_deg_kernel(ch):
  nch = EPW // ch

  @functools.partial(
      pl.kernel, mesh=_MESH,
      compiler_params=_SC_PARAMS,
      out_type=jax.ShapeDtypeStruct((NC, N, 16), jnp.float32),
      scratch_types=[
          pltpu.VMEM((nch, ch), jnp.int32),
          pltpu.VMEM((ch, 16), jnp.float32),
          pltpu.VMEM_SHARED((N, 16), jnp.float32),
      ],
  )
  def deg_kernel(dst_hbm, zeros_hbm, out_hbm, idx_v, ones_v, deg_sh):
    cid = lax.axis_index("c")
    sid = lax.axis_index("s")
    wid = cid * NS + sid
    pltpu.sync_copy(zeros_hbm, deg_sh.at[pl.ds(sid * RPS, RPS)])
    pltpu.sync_copy(dst_hbm.at[wid], idx_v)
    for r in range(ch):
      ones_v[r, :] = jnp.full((16,), 1.0, jnp.float32)
    plsc.subcore_barrier()

    def body(c, carry):
      pltpu.sync_copy(ones_v, deg_sh.at[idx_v.at[c]], add=True)
      return carry

    lax.fori_loop(0, nch, body, 0)
    plsc.subcore_barrier()
    pltpu.sync_copy(deg_sh.at[pl.ds(sid * RPS, RPS)],
                    out_hbm.at[cid, pl.ds(sid * RPS, RPS)])

  return deg_kernel


def _make_edge_kernel(d, ch):
  nch = EPW // ch

  @functools.partial(
      pl.kernel, mesh=_MESH,
      compiler_params=_SC_PARAMS,
      out_type=jax.ShapeDtypeStruct((NC, N, d), jnp.float32),
      scratch_types=[
          pltpu.VMEM((nch, ch), jnp.int32),
          pltpu.VMEM((nch, ch), jnp.int32),
          pltpu.VMEM((ch, d), jnp.float32),
          pltpu.VMEM((ch, d), jnp.float32),
          pltpu.VMEM((ch, d), jnp.float32),
          pltpu.VMEM((ch, d), jnp.float32),
          pltpu.VMEM_SHARED((N, d), jnp.float32),
          pltpu.SemaphoreType.DMA,
          pltpu.SemaphoreType.DMA,
      ],
  )
  def edge_kernel(y_hbm, src_hbm, dst_hbm, zeros_hbm, out_hbm,
                  src_v, dst_v, rows0, rows1, rows2, rows3, acc_sh,
                  sem_g, sem_s):
    cid = lax.axis_index("c")
    sid = lax.axis_index("s")
    wid = cid * NS + sid
    pltpu.sync_copy(zeros_hbm, acc_sh.at[pl.ds(sid * RPS, RPS)])
    pltpu.sync_copy(src_hbm.at[wid], src_v)
    pltpu.sync_copy(dst_hbm.at[wid], dst_v)
    plsc.subcore_barrier()

    bufs = (rows0, rows1, rows2, rows3)
    pltpu.async_copy(y_hbm.at[src_v.at[0]], bufs[0], sem_g)
    pltpu.async_copy(y_hbm.at[src_v.at[1]], bufs[1], sem_g)

    def step(c, b):
      @pl.when(c >= 2)
      def _():
        pltpu.make_async_copy(bufs[(b + 2) % 4],
                              acc_sh.at[dst_v.at[c - 2]], sem_s).wait()

      @pl.when(c + 2 < nch)
      def _():
        pltpu.async_copy(y_hbm.at[src_v.at[c + 2]], bufs[(b + 2) % 4], sem_g)

      pltpu.make_async_copy(y_hbm.at[src_v.at[c]], bufs[b], sem_g).wait()
      pltpu.async_copy(bufs[b], acc_sh.at[dst_v.at[c]], sem_s, add=True)

    def body(k, carry):
      c = k * 4
      for off in range(4):
        step(c + off, off)
      return carry

    lax.fori_loop(0, nch // 4, body, 0)
    for c in range((nch // 4) * 4, nch):
      step(c, c % 4)
    pltpu.make_async_copy(bufs[(nch - 2) % 4],
                          acc_sh.at[dst_v.at[nch - 2]], sem_s).wait()
    pltpu.make_async_copy(bufs[(nch - 1) % 4],
                          acc_sh.at[dst_v.at[nch - 1]], sem_s).wait()
    plsc.subcore_barrier()
    pltpu.sync_copy(acc_sh.at[pl.ds(sid * RPS, RPS)],
                    out_hbm.at[cid, pl.ds(sid * RPS, RPS)])

  return edge_kernel


def _make_decode_kernel(d, ch):
  nch = EPW // ch

  nbuf = 3

  @functools.partial(
      pl.kernel, mesh=_MESH,
      compiler_params=_SC_PARAMS,
      out_type=jax.ShapeDtypeStruct((NW, nch, ch), jnp.float32),
      scratch_types=[
          pltpu.VMEM((nch, ch), jnp.int32),
          pltpu.VMEM((nch, ch), jnp.int32),
          [pltpu.VMEM((ch, d), jnp.float32) for _ in range(nbuf)],
          [pltpu.VMEM((ch, d), jnp.float32) for _ in range(nbuf)],
          pltpu.VMEM((nch, ch), jnp.float32),
          pltpu.SemaphoreType.DMA,
      ],
  )
  def decode_kernel(z_hbm, src_hbm, dst_hbm, out_hbm,
                    src_v, dst_v, zrs, zcs, out_v, sem_g):
    cid = lax.axis_index("c")
    sid = lax.axis_index("s")
    wid = cid * NS + sid
    pltpu.sync_copy(src_hbm.at[wid], src_v)
    pltpu.sync_copy(dst_hbm.at[wid], dst_v)

    for p in range(nbuf - 1):
      pltpu.async_copy(z_hbm.at[src_v.at[p]], zrs[p], sem_g)
      pltpu.async_copy(z_hbm.at[dst_v.at[p]], zcs[p], sem_g)

    lanes = lax.iota(jnp.int32, 16)

    def step(c, b):
      zr_cur, zc_cur = zrs[b], zcs[b]
      nb = (b + nbuf - 1) % nbuf

      @pl.when(c + nbuf - 1 < nch)
      def _():
        pltpu.async_copy(z_hbm.at[src_v.at[c + nbuf - 1]], zrs[nb], sem_g)
        pltpu.async_copy(z_hbm.at[dst_v.at[c + nbuf - 1]], zcs[nb], sem_g)

      pltpu.make_async_copy(z_hbm.at[src_v.at[c]], zr_cur, sem_g).wait()
      pltpu.make_async_copy(z_hbm.at[dst_v.at[c]], zc_cur, sem_g).wait()
      def group(g, carry):
        rows = lanes + g * 16
        accs = [jnp.zeros((16,), jnp.float32) for _ in range(4)]
        for t in range(16):
          colt = (lanes + t) & 15
          for jb in range(d // 16):
            col = colt + (jb * 16)
            accs[jb] = accs[jb] + (plsc.load_gather(zr_cur, [rows, col]) *
                                   plsc.load_gather(zc_cur, [rows, col]))
        out_v[c, pl.ds(g * 16, 16)] = (accs[0] + accs[1]) + (accs[2] + accs[3])
        return carry

      lax.fori_loop(0, ch // 16, group, 0)

    def body(k, carry):
      c = k * nbuf
      for off in range(nbuf):
        step(c + off, off)
      return carry

    lax.fori_loop(0, nch // nbuf, body, 0)
    for c in range((nch // nbuf) * nbuf, nch):
      step(c, c % nbuf)
    pltpu.sync_copy(out_v, out_hbm.at[wid])

  return decode_kernel


def _tc_y1(x, W1, deg0, deg1):
  blk = 1000
  grid = N // blk

  def body(xb, wb, d0b, d1b, yb):
    dinv = lax.rsqrt(d0b[...] + d1b[...] + 1.0)
    yb[...] = dinv * jnp.dot(xb[...], wb[...],
                             preferred_element_type=jnp.float32)

  return pl.pallas_call(
      body,
      grid=(grid,),
      in_specs=[
          pl.BlockSpec((blk, 128), lambda i: (i, 0)),
          pl.BlockSpec((128, 128), lambda i: (0, 0)),
          pl.BlockSpec((blk, 1), lambda i: (i, 0)),
          pl.BlockSpec((blk, 1), lambda i: (i, 0)),
      ],
      out_specs=pl.BlockSpec((blk, 128), lambda i: (i, 0)),
      out_shape=jax.ShapeDtypeStruct((N, 128), jnp.float32),
  )(x, W1, deg0, deg1)


def _tc_layer2_in(S1p, y1, deg0, deg1, b1, W2):
  blk = 1000
  grid = N // blk

  def body(sb, yb, d0b, d1b, bb, wb, ob):
    dinv = lax.rsqrt(d0b[...] + d1b[...] + 1.0)
    h = jax.nn.relu(dinv * (sb[0] + sb[1] + yb[...]) + bb[...])
    ob[...] = dinv * jnp.dot(h, wb[...], preferred_element_type=jnp.float32)

  return pl.pallas_call(
      body,
      grid=(grid,),
      in_specs=[
          pl.BlockSpec((2, blk, 128), lambda i: (0, i, 0)),
          pl.BlockSpec((blk, 128), lambda i: (i, 0)),
          pl.BlockSpec((blk, 1), lambda i: (i, 0)),
          pl.BlockSpec((blk, 1), lambda i: (i, 0)),
          pl.BlockSpec((1, 128), lambda i: (0, 0)),
          pl.BlockSpec((128, 64), lambda i: (0, 0)),
      ],
      out_specs=pl.BlockSpec((blk, 64), lambda i: (i, 0)),
      out_shape=jax.ShapeDtypeStruct((N, 64), jnp.float32),
  )(S1p, y1, deg0, deg1, b1, W2)


def _tc_z(S2p, y2, deg0, deg1, b2):
  blk = 1000
  grid = N // blk

  def body(sb, yb, d0b, d1b, bb, zb):
    dinv = lax.rsqrt(d0b[...] + d1b[...] + 1.0)
    zb[...] = dinv * (sb[0] + sb[1] + yb[...]) + bb[...]

  return pl.pallas_call(
      body,
      grid=(grid,),
      in_specs=[
          pl.BlockSpec((2, blk, 64), lambda i: (0, i, 0)),
          pl.BlockSpec((blk, 64), lambda i: (i, 0)),
          pl.BlockSpec((blk, 1), lambda i: (i, 0)),
          pl.BlockSpec((blk, 1), lambda i: (i, 0)),
          pl.BlockSpec((1, 64), lambda i: (0, 0)),
      ],
      out_specs=pl.BlockSpec((blk, 64), lambda i: (i, 0)),
      out_shape=jax.ShapeDtypeStruct((N, 64), jnp.float32),
  )(S2p, y2, deg0, deg1, b2)


_deg_kernel = _make_deg_kernel(80)
_edge_kernel_128 = _make_edge_kernel(128, 40)
_edge_kernel_64 = _make_edge_kernel(64, 80)
_decode_kernel = _make_decode_kernel(64, 80)


def kernel(x, edge_index, W1, b1, W2, b2):
  e = edge_index.shape[1]
  src80 = edge_index[0].reshape(NW, EPW // 80, 80)
  dst80 = edge_index[1].reshape(NW, EPW // 80, 80)
  src40 = edge_index[0].reshape(NW, EPW // 40, 40)
  dst40 = edge_index[1].reshape(NW, EPW // 40, 40)

  deg_part = _deg_kernel(dst80, jnp.zeros((RPS, 16), jnp.float32))
  deg0 = deg_part[0, :, 0:1]
  deg1 = deg_part[1, :, 0:1]

  y1 = _tc_y1(x, W1, deg0, deg1)
  S1p = _edge_kernel_128(y1, src40, dst40, jnp.zeros((RPS, 128), jnp.float32))
  y2 = _tc_layer2_in(S1p, y1, deg0, deg1, b1.reshape(1, 128), W2)
  S2p = _edge_kernel_64(y2, src80, dst80, jnp.zeros((RPS, 64), jnp.float32))
  z = _tc_z(S2p, y2, deg0, deg1, b2.reshape(1, 64))

  scores = _decode_kernel(z, src80, dst80)
  return scores.reshape(e)
